# baseline (speedup 1.0000x reference)
import functools

import jax
import jax.numpy as jnp
from jax import lax
from jax.experimental import pallas as pl
from jax.experimental.pallas import tpu as pltpu

N_EDGE = 320000
N_TRIP = 960000
H = 128
INT = 64
NB = 6
NCLS = 5

CHUNK = 2048
NBKT = (N_EDGE + CHUNK - 1) // CHUNK
PAD_EDGE = NBKT * CHUNK


def _silu(v):
    return v * jax.nn.sigmoid(v)


def _bmm(u, w):
    return jnp.dot(u.astype(jnp.bfloat16), w.astype(jnp.bfloat16),
                   preferred_element_type=jnp.float32)


def _t1_body(x_ref, rbf_ref, wkj_ref, bkj_ref, wr1_ref, wr2_ref, wd_ref,
             out_ref):
    xb = x_ref[...]
    r = jnp.dot(jnp.dot(rbf_ref[...], wr1_ref[...],
                        preferred_element_type=jnp.float32), wr2_ref[...],
                preferred_element_type=jnp.float32)
    t = _silu(_bmm(xb, wkj_ref[...]) + bkj_ref[...]) * r
    res = _silu(_bmm(t, wd_ref[...]))
    out_ref[...] = jnp.concatenate([res, jnp.zeros_like(res)], axis=1)


def _t1(x, rbf, W_kj, b_kj, W_rbf1, W_rbf2, W_down):
    B = 2048
    grid = (pl.cdiv(N_EDGE, B),)
    return pl.pallas_call(
        _t1_body,
        grid=grid,
        in_specs=[
            pl.BlockSpec((B, H), lambda i: (i, 0)),
            pl.BlockSpec((B, 8), lambda i: (i, 0)),
            pl.BlockSpec((H, H), lambda i: (0, 0)),
            pl.BlockSpec((1, H), lambda i: (0, 0)),
            pl.BlockSpec((8, 8), lambda i: (0, 0)),
            pl.BlockSpec((8, H), lambda i: (0, 0)),
            pl.BlockSpec((H, INT), lambda i: (0, 0)),
        ],
        out_specs=pl.BlockSpec((B, 2 * INT), lambda i: (i, 0)),
        out_shape=jax.ShapeDtypeStruct((N_EDGE, 2 * INT), jnp.float32),
    )(x, _pad_minor(rbf, 8), W_kj, b_kj.reshape(1, H),
      _pad_rows(W_rbf1, 8), W_rbf2, W_down)


def _pad_minor(a, to):
    if a.shape[-1] == to:
        return a
    return jnp.pad(a, ((0, 0), (0, to - a.shape[-1])))


def _pad_rows(a, to):
    if a.shape[0] == to:
        return a
    return jnp.pad(a, ((0, to - a.shape[0]), (0, 0)))


def _t2_body(sbf_ref, ws1_ref, ws2_ref, out_ref):
    t = jnp.dot(sbf_ref[...], ws1_ref[...],
                preferred_element_type=jnp.float32)
    res = jnp.dot(t, ws2_ref[...], preferred_element_type=jnp.float32)
    out_ref[...] = jnp.concatenate([res, jnp.zeros_like(res)], axis=1)


def _t2(sbf, W_sbf1, W_sbf2):
    B = 4096
    K = 48
    return pl.pallas_call(
        _t2_body,
        grid=(pl.cdiv(N_TRIP, B),),
        in_specs=[
            pl.BlockSpec((B, K), lambda i: (i, 0)),
            pl.BlockSpec((K, 8), lambda i: (0, 0)),
            pl.BlockSpec((8, INT), lambda i: (0, 0)),
        ],
        out_specs=pl.BlockSpec((B, 2 * INT), lambda i: (i, 0)),
        out_shape=jax.ShapeDtypeStruct((N_TRIP, 2 * INT), jnp.float32),
    )(_pad_minor(sbf, K), _pad_rows(W_sbf1, K), W_sbf2)


def _t3_body(x_ref, xs_ref, alpha_ref, wji_ref, bji_ref, wup_ref, wb1_ref,
             bb1_ref, wb2_ref, bb2_ref, wlin_ref, blin_ref, wa1_ref, ba1_ref,
             wa2_ref, ba2_ref, out_ref):
    xb = x_ref[...]
    a = alpha_ref[0, 0]

    mm = _bmm

    def branch(idx, inp):
        if inp is None:
            u = jnp.zeros((xb.shape[0], H), jnp.float32)
        else:
            u = _silu(mm(inp, wup_ref[idx]))
        h = _silu(mm(xb, wji_ref[idx]) + bji_ref[idx]) + u
        h = h + _silu(mm(_silu(mm(h, wb1_ref[idx]) + bb1_ref[idx]),
                         wb2_ref[idx]) + bb2_ref[idx])
        h = _silu(mm(h, wlin_ref[idx]) + blin_ref[idx]) + xb
        h = h + _silu(mm(_silu(mm(h, wa1_ref[idx]) + ba1_ref[idx]),
                         wa2_ref[idx]) + ba2_ref[idx])
        return h

    g = xs_ref[0] + xs_ref[1] + xs_ref[2] + xs_ref[3] + xs_ref[4]
    acc = a * branch(NB - 1, g)
    acc = acc + (1.0 - a) * branch(0, None)
    for b in range(1, NB):
        acc = acc + (1.0 - a) * branch(b, xs_ref[b - 1])
    out_ref[...] = acc


def _t3(x, xsum, alpha, W_ji, b_ji, W_up, Wb1, bb1, Wb2, bb2, W_lin, b_lin,
        Wa1, ba1, Wa2, ba2):
    B = 1024
    wspec = pl.BlockSpec((NB, H, H), lambda i: (0, 0, 0))
    bspec = pl.BlockSpec((NB, 1, H), lambda i: (0, 0, 0))
    return pl.pallas_call(
        _t3_body,
        grid=(pl.cdiv(N_EDGE, B),),
        in_specs=[
            pl.BlockSpec((B, H), lambda i: (i, 0)),
            pl.BlockSpec((NCLS, B, INT), lambda i: (0, i, 0)),
            pl.BlockSpec(memory_space=pltpu.SMEM),
            wspec, bspec,
            pl.BlockSpec((NB, INT, H), lambda i: (0, 0, 0)),
            wspec, bspec, wspec, bspec, wspec, bspec, wspec, bspec,
            wspec, bspec,
        ],
        out_specs=pl.BlockSpec((B, H), lambda i: (i, 0)),
        out_shape=jax.ShapeDtypeStruct((N_EDGE, H), jnp.float32),
    )(x, xsum, alpha.reshape(1, 1), W_ji, b_ji.reshape(NB, 1, H), W_up,
      Wb1, bb1.reshape(NB, 1, H), Wb2, bb2.reshape(NB, 1, H),
      W_lin, b_lin.reshape(NB, 1, H), Wa1, ba1.reshape(NB, 1, H),
      Wa2, ba2.reshape(NB, 1, H))


from jax.experimental.pallas import tpu_sc as plsc

NC = 2
NS = 16
NWORK = NC * NS
GSIZE = 128
GROUPS = N_TRIP // GSIZE
BASE_G = GROUPS // NWORK
EXTRA_G = GROUPS - BASE_G * NWORK
REGION_CAP = 31184
PERM_TOTAL = NWORK * REGION_CAP + GSIZE
ACC_ROWS = NCLS * CHUNK + 256
TRASH_ROW = NCLS * CHUNK


def _iota16():
    return lax.iota(jnp.int32, 16)


def _splat(v):
    return jnp.broadcast_to(v, (16,))


def _sc_bin_body(ji_hbm, kj_hbm, bt_hbm, perm_t, perm_kj, perm_row,
                 ptr_hbm, cnt_hbm,
                 ji_st, kj_st, bt_st, hist2d, histv, ptrv, curv, tmp48,
                 tmp16, pos_st, t_st, row_st, sem):
    c = lax.axis_index("c")
    s = lax.axis_index("s")
    w = s * NC + c
    ng = BASE_G + jnp.where(w < EXTRA_G, 1, 0)
    iota = _iota16()
    ones = jnp.full((16,), 1, jnp.int32)

    def _zero_hist(i, _):
        hist2d[pl.ds(i * 16, 16)] = jnp.zeros((16,), jnp.int32)
        return 0
    lax.fori_loop(0, 160, _zero_hist, 0)

    def _hist_group(i, _):
        g = w + i * NWORK
        pltpu.sync_copy(ji_hbm.at[g], ji_st)
        for v in range(8):
            ji = ji_st[pl.ds(v * 16, 16)]
            b = lax.shift_right_logical(ji, 11)
            plsc.addupdate_scatter(hist2d, [iota * 160 + b], ones)
        return 0
    lax.fori_loop(0, ng, _hist_group, 0)

    run = w * REGION_CAP
    for bg in range(10):
        cnt16 = jnp.zeros((16,), jnp.int32)
        for j in range(16):
            cnt16 = cnt16 + hist2d[pl.ds(j * 160 + bg * 16, 16)]
        histv[pl.ds(bg * 16, 16)] = cnt16
        p16 = jnp.bitwise_and(cnt16 + 7, -8)
        incl = plsc.cumsum(p16)
        ptrv[pl.ds(bg * 16, 16)] = incl - p16 + run
        curv[pl.ds(bg * 16, 16)] = incl - p16 + run
        run = run + jnp.max(incl)
    pltpu.sync_copy(histv, cnt_hbm.at[w])
    pltpu.sync_copy(ptrv, ptr_hbm.at[w])

    tmp48[pl.ds(0, 16)] = jnp.full((16,), -1, jnp.int32)
    tmp48[pl.ds(32, 16)] = jnp.full((16,), -2, jnp.int32)

    def _perm_group(i, _):
        g = w + i * NWORK
        pltpu.sync_copy(ji_hbm.at[g], ji_st)
        pltpu.sync_copy(kj_hbm.at[g], kj_st)
        pltpu.async_copy(bt_hbm.at[kj_st], bt_st, sem).wait()
        for v in range(8):
            sl = pl.ds(v * 16, 16)
            ji = ji_st[sl]
            btv = bt_st[sl]
            b = lax.shift_right_logical(ji, 11)
            skey, perm = plsc.sort_key_val(b, iota)
            tmp48[pl.ds(16, 16)] = skey
            prev = plsc.load_gather(tmp48, [iota + 15])
            nxt = plsc.load_gather(tmp48, [iota + 17])
            m_start = skey != prev
            rank = iota - plsc.cummax(jnp.where(m_start, iota, 0))
            pos_sorted = plsc.load_gather(curv, [skey]) + rank
            plsc.addupdate_scatter(curv, [skey], rank + 1, mask=skey != nxt)
            plsc.store_scatter(tmp16, [perm], pos_sorted)
            pos_st[sl] = tmp16[pl.ds(0, 16)]
            row_st[sl] = jnp.bitwise_or(lax.shift_left(btv, 11),
                                        jnp.bitwise_and(ji, 2047))
            t_st[sl] = g * GSIZE + v * 16 + iota
        d1 = pltpu.async_copy(t_st, perm_t.at[pos_st], sem)
        d2 = pltpu.async_copy(kj_st, perm_kj.at[pos_st], sem)
        d3 = pltpu.async_copy(row_st, perm_row.at[pos_st], sem)
        d1.wait()
        d2.wait()
        d3.wait()
        return 0
    lax.fori_loop(0, ng, _perm_group, 0)


def _sc_bin(ji2d, kj2d, bt):
    i32 = jnp.int32
    return pl.kernel(
        _sc_bin_body,
        out_type=(
            jax.ShapeDtypeStruct((PERM_TOTAL,), i32),
            jax.ShapeDtypeStruct((PERM_TOTAL,), i32),
            jax.ShapeDtypeStruct((PERM_TOTAL,), i32),
            jax.ShapeDtypeStruct((NWORK, 160), i32),
            jax.ShapeDtypeStruct((NWORK, 160), i32),
        ),
        mesh=plsc.VectorSubcoreMesh(core_axis_name="c", subcore_axis_name="s"),
        compiler_params=pltpu.CompilerParams(needs_layout_passes=False),
        scratch_types=[
            pltpu.VMEM((GSIZE,), i32),
            pltpu.VMEM((GSIZE,), i32),
            pltpu.VMEM((GSIZE,), i32),
            pltpu.VMEM((2560,), i32),
            pltpu.VMEM((160,), i32),
            pltpu.VMEM((160,), i32),
            pltpu.VMEM((160,), i32),
            pltpu.VMEM((48,), i32),
            pltpu.VMEM((16,), i32),
            pltpu.VMEM((GSIZE,), i32),
            pltpu.VMEM((GSIZE,), i32),
            pltpu.VMEM((GSIZE,), i32),
            pltpu.SemaphoreType.DMA,
        ],
    )(ji2d, kj2d, bt)


def _sc_acc_body(perm_t, perm_kj, perm_row, ptr_hbm, cnt_hbm, xkj_hbm,
                 s2_hbm, xsum_hbm,
                 acc, t_st, kj_st, row_st, xv, sv, pv, ptr_st, cnt_st,
                 zero_st, sem):
    c = lax.axis_index("c")
    s = lax.axis_index("s")
    iota = _iota16()

    def _zb(i, _):
        for q in range(INT // 16):
            zero_st[i, pl.ds(q * 16, 16)] = jnp.zeros((16,), jnp.float32)
        return 0
    lax.fori_loop(0, 256, _zb, 0)
    zero2d = zero_st

    nrounds = (NBKT + NC - 1) // NC

    def _round(i, _):
        k = i * NC + c

        @pl.when(k < NBKT)
        def _():
            for j in range(3):
                blk = s + j * NS

                @pl.when(blk < ACC_ROWS // 256)
                def _():
                    pltpu.sync_copy(zero2d, acc.at[pl.ds(blk * 256, 256)])
            plsc.subcore_barrier()

            for rj in range(2):
                r = s * 2 + rj
                pltpu.sync_copy(ptr_hbm.at[r], ptr_st)
                pltpu.sync_copy(cnt_hbm.at[r], cnt_st)
                kvec = _splat(k)
                start = jnp.max(plsc.load_gather(ptr_st, [kvec]))
                endp = jnp.max(plsc.load_gather(ptr_st, [kvec + 1]))
                nreal = jnp.max(plsc.load_gather(cnt_st, [kvec]))
                nwin = lax.shift_right_logical(endp - start + 127, 7)

                def _win(wi, _):
                    base = pl.multiple_of(start + wi * GSIZE, 8)
                    pltpu.sync_copy(perm_t.at[pl.ds(base, GSIZE)], t_st)
                    pltpu.sync_copy(perm_kj.at[pl.ds(base, GSIZE)], kj_st)
                    pltpu.sync_copy(perm_row.at[pl.ds(base, GSIZE)], row_st)
                    for v in range(8):
                        sl = pl.ds(v * 16, 16)
                        o = wi * GSIZE + v * 16 + iota
                        m = o < _splat(nreal)
                        t_st[sl] = jnp.where(m, t_st[sl], 0)
                        kj_st[sl] = jnp.where(m, kj_st[sl], 0)
                        row_st[sl] = jnp.where(m, row_st[sl],
                                               TRASH_ROW + iota)
                    g1 = pltpu.async_copy(xkj_hbm.at[kj_st], xv, sem)
                    g1.wait()
                    g2 = pltpu.async_copy(s2_hbm.at[t_st], sv, sem)
                    g2.wait()

                    def _mul(ri, _):
                        for q in range(4):
                            qs = pl.ds(q * 16, 16)
                            pv[ri, qs] = xv[ri, qs] * sv[ri, qs]
                        return 0
                    lax.fori_loop(0, GSIZE, _mul, 0)
                    pltpu.sync_copy(pv, acc.at[row_st], add=True)
                    return 0
                lax.fori_loop(0, nwin, _win, 0)
            plsc.subcore_barrier()

            for cc in range(NCLS):
                src = acc.at[pl.ds(cc * CHUNK + s * 128, 128)]
                dst = xsum_hbm.at[pl.ds(cc * PAD_EDGE + k * CHUNK + s * 128,
                                        128)]
                pltpu.sync_copy(src, dst)
            plsc.subcore_barrier()
        return 0
    lax.fori_loop(0, nrounds, _round, 0)


def _sc_acc(perm_t, perm_kj, perm_row, ptr, cnt, x_kj64, s2):
    i32 = jnp.int32
    f32 = jnp.float32
    return pl.kernel(
        _sc_acc_body,
        out_type=jax.ShapeDtypeStruct((NCLS * PAD_EDGE, INT), f32),
        mesh=plsc.VectorSubcoreMesh(core_axis_name="c", subcore_axis_name="s"),
        compiler_params=pltpu.CompilerParams(needs_layout_passes=False),
        scratch_types=[
            pltpu.VMEM_SHARED((ACC_ROWS, INT), f32),
            pltpu.VMEM((GSIZE,), i32),
            pltpu.VMEM((GSIZE,), i32),
            pltpu.VMEM((GSIZE,), i32),
            pltpu.VMEM((GSIZE, 2 * INT), f32),
            pltpu.VMEM((GSIZE, 2 * INT), f32),
            pltpu.VMEM((GSIZE, INT), f32),
            pltpu.VMEM((160,), i32),
            pltpu.VMEM((160,), i32),
            pltpu.VMEM((256, INT), f32),
            pltpu.SemaphoreType.DMA,
        ],
    )(perm_t, perm_kj, perm_row, ptr, cnt, x_kj64, s2)


def _sparse_xsum(x_kj64, s2, idx_kj, idx_ji, bt):
    ji2d = idx_ji.reshape(GROUPS, GSIZE)
    kj2d = idx_kj.reshape(GROUPS, GSIZE)
    perm_t, perm_kj, perm_row, ptr, cnt = _sc_bin(ji2d, kj2d, bt)
    out = _sc_acc(perm_t, perm_kj, perm_row, ptr, cnt, x_kj64, s2)
    return out.reshape(NCLS, PAD_EDGE, INT)


def kernel(x, rbf, sbf, alpha, lambda_d, W_rbf1, W_rbf2, W_sbf1, W_sbf2,
           W_kj, b_kj, W_ji, b_ji, W_down, W_up, Wb1, bb1, Wb2, bb2,
           W_lin, b_lin, Wa1, ba1, Wa2, ba2, idx_kj, idx_ji, bt):
    x_kj64 = _t1(x, rbf, W_kj, b_kj, W_rbf1, W_rbf2, W_down)
    s2 = _t2(sbf, W_sbf1, W_sbf2)
    xsum = _sparse_xsum(x_kj64, s2, idx_kj, idx_ji, bt)
    return _t3(x, xsum, alpha, W_ji, b_ji, W_up, Wb1, bb1, Wb2, bb2,
               W_lin, b_lin, Wa1, ba1, Wa2, ba2)
